# Initial kernel scaffold; baseline (speedup 1.0000x reference)
import jax
import jax.numpy as jnp
from jax.experimental import pallas as pl


def kernel(logits, target):
    raise NotImplementedError("write your pallas kernel here")


# speedup vs baseline: 1.9581x; 1.9581x over previous
import jax
import jax.numpy as jnp
from jax.experimental import pallas as pl

_ROW_BLOCK = 32


def _margin_kernel(t_ref, x_ref, o_ref):
    x = x_ref[...]
    t = t_ref[...]
    col = jax.lax.broadcasted_iota(jnp.int32, x.shape, 1)
    eq = col == t
    neg = jnp.float32(-jnp.inf)
    masked_max = jnp.max(jnp.where(eq, neg, x), axis=1, keepdims=True)
    true_val = jnp.max(jnp.where(eq, x, neg), axis=1, keepdims=True)
    o_ref[...] = masked_max - true_val


def kernel(logits, target):
    B, V = logits.shape
    R = _ROW_BLOCK
    t2 = target.astype(jnp.int32).reshape(B, 1)
    out = pl.pallas_call(
        _margin_kernel,
        grid=(B // R,),
        in_specs=[
            pl.BlockSpec((R, 1), lambda i: (i, 0)),
            pl.BlockSpec((R, V), lambda i: (i, 0)),
        ],
        out_specs=pl.BlockSpec((R, 1), lambda i: (i, 0)),
        out_shape=jax.ShapeDtypeStruct((B, 1), jnp.float32),
    )(t2, logits)
    return out.reshape(B)


# speedup vs baseline: 1.9607x; 1.0013x over previous
import jax
import jax.numpy as jnp
from jax.experimental import pallas as pl

_ROW_BLOCK = 64


def _margin_kernel(t_ref, x_ref, o_ref):
    x = x_ref[...]
    t = t_ref[...]
    col = jax.lax.broadcasted_iota(jnp.int32, x.shape, 1)
    eq = col == t
    neg = jnp.float32(-jnp.inf)
    masked_max = jnp.max(jnp.where(eq, neg, x), axis=1, keepdims=True)
    true_val = jnp.max(jnp.where(eq, x, neg), axis=1, keepdims=True)
    o_ref[...] = masked_max - true_val


def kernel(logits, target):
    B, V = logits.shape
    R = _ROW_BLOCK
    t2 = target.astype(jnp.int32).reshape(B, 1)
    out = pl.pallas_call(
        _margin_kernel,
        grid=(B // R,),
        in_specs=[
            pl.BlockSpec((R, 1), lambda i: (i, 0)),
            pl.BlockSpec((R, V), lambda i: (i, 0)),
        ],
        out_specs=pl.BlockSpec((R, 1), lambda i: (i, 0)),
        out_shape=jax.ShapeDtypeStruct((B, 1), jnp.float32),
    )(t2, logits)
    return out.reshape(B)
